# Initial kernel scaffold; baseline (speedup 1.0000x reference)
import jax
import jax.numpy as jnp
from jax.experimental import pallas as pl


def kernel(x, indices, W, b):
    raise NotImplementedError("write your pallas kernel here")


# speedup vs baseline: 1.8139x; 1.8139x over previous
import functools

import jax
import jax.numpy as jnp
from jax import lax
from jax.experimental import pallas as pl
from jax.experimental.pallas import tpu as pltpu
from jax.experimental.pallas import tpu_sc as plsc

N_NODES = 50000
D = 128
S = 9
O = 128

NC = 2
NS = 16
L = 16
NW = NC * NS

CH = 32
PER_W = 1568
NCHUNK = PER_W // CH
NPAD = NW * PER_W
ROWS = CH * S
GR = 96
NG = ROWS // GR

BR = 512


def _matmul_body(x_ref, w_ref, z_ref):
    z_ref[...] = jnp.dot(x_ref[...], w_ref[...],
                         preferred_element_type=jnp.float32)


def _tc_matmul(x, wcat):
    n = x.shape[0]
    grid = (pl.cdiv(n, BR),)
    return pl.pallas_call(
        _matmul_body,
        grid=grid,
        in_specs=[
            pl.BlockSpec((BR, D), lambda i: (i, 0)),
            pl.BlockSpec((D, S * O), lambda i: (0, 0)),
        ],
        out_specs=pl.BlockSpec((BR, S * O), lambda i: (i, 0)),
        out_shape=jax.ShapeDtypeStruct((n, S * O), jnp.float32),
    )(x, wcat)


def _sc_body(z_ref, idx_ref, b_ref, out_ref,
             rawbuf, fbuf0, fbuf1, fbuf2, gbuf, obuf, bbuf, sem):
    fbufs = (fbuf0, fbuf1, fbuf2)
    wid = lax.axis_index("s") * NC + lax.axis_index("c")
    base = wid * PER_W

    pltpu.sync_copy(b_ref, bbuf)
    bvecs = [bbuf[pl.ds(p * L, L)] for p in range(O // L)]

    def chunk_body(k, carry):
        dbase = base + k * CH
        pltpu.sync_copy(idx_ref.at[pl.ds(dbase * S, ROWS)], rawbuf)
        for c in range(ROWS // L):
            jv = lax.iota(jnp.int32, L) + (c * L)
            sv = lax.rem(jv, S)
            fv = rawbuf[pl.ds(c * L, L)] * S + sv
            fbufs[c // (GR // L)][pl.ds((c % (GR // L)) * L, L)] = fv
        for g in range(NG):
            pltpu.async_copy(z_ref.at[fbufs[g]],
                             gbuf.at[pl.ds(g * GR, GR)], sem).wait()

        def acc_body(n, c2):
            accs = list(bvecs)
            for s in range(S):
                row = n * S + s
                for p in range(O // L):
                    accs[p] = accs[p] + gbuf[row, pl.ds(p * L, L)]
            for p in range(O // L):
                obuf[n, pl.ds(p * L, L)] = accs[p]
            return c2

        lax.fori_loop(0, CH, acc_body, 0)
        pltpu.sync_copy(obuf, out_ref.at[pl.ds(dbase, CH)])
        return carry

    lax.fori_loop(0, NCHUNK, chunk_body, 0)


_sc_gather_sum = functools.partial(
    pl.kernel,
    out_type=jax.ShapeDtypeStruct((NPAD, O), jnp.float32),
    mesh=plsc.VectorSubcoreMesh(core_axis_name="c", subcore_axis_name="s",
                                num_cores=NC, num_subcores=NS),
    scratch_types=[
        pltpu.VMEM((ROWS,), jnp.int32),
        pltpu.VMEM((GR,), jnp.int32),
        pltpu.VMEM((GR,), jnp.int32),
        pltpu.VMEM((GR,), jnp.int32),
        pltpu.VMEM((ROWS, O), jnp.float32),
        pltpu.VMEM((CH, O), jnp.float32),
        pltpu.VMEM((O,), jnp.float32),
        pltpu.SemaphoreType.DMA,
    ],
)(_sc_body)


def kernel(x, indices, W, b):
    n_nodes = x.shape[0]
    wcat = W.reshape(S, D, O).transpose(1, 0, 2).reshape(D, S * O)
    z = _tc_matmul(x, wcat)
    zflat = z.reshape(n_nodes * S, O)
    idx_pad = jnp.pad(indices, ((0, NPAD - n_nodes), (0, 0))).reshape(-1)
    out = _sc_gather_sum(zflat, idx_pad.astype(jnp.int32), b)
    return out[:n_nodes]
